# baseline (speedup 1.0000x reference)
import functools
import numpy as np
import jax
import jax.numpy as jnp
from jax import lax
from jax.experimental import pallas as pl
from jax.experimental.pallas import tpu as pltpu
from jax.experimental.pallas import tpu_sc as plsc

_NC, _NS = 2, 16
_NW = _NC * _NS
_P = 128
_CH = 512
_NCH = max(1, 8 * _P // _CH)

_P1, _P2, _P3 = 2654435761, 29675113, 123456789


def _c32(v):
    return jnp.int32(np.int32(np.uint32(v & 0xFFFFFFFF)))


_HA = (_P1 * _P2 * _P3) & 0xFFFFFFFF
_HB = (_P2 * _P3) & 0xFFFFFFFF
_HC = _P3 & 0xFFFFFFFF
_OFFS = [((k & 1) * _HA + ((k >> 1) & 1) * _HB + ((k >> 2) & 1) * _HC) & 0xFFFFFFFF
         for k in range(8)]

_N_VMEM_LEV = 14
_BASE0 = [2 ** (l + 2) - 4 for l in range(16)]
_BASE1 = [_BASE0[l] + 2 ** (l + 1) for l in range(16)]
_TAB_WORDS = 2 ** 16


def _weights(tx, ty, tz):
    u = 1.0 - tx
    v = 1.0 - ty
    w = 1.0 - tz
    w00 = u * v
    w10 = tx * v
    w01 = u * ty
    w11 = tx * ty
    return [w00 * w, w10 * w, w01 * w, w11 * w,
            w00 * tz, w10 * tz, w01 * tz, w11 * tz]


def _floor_frac(c, res):
    s = (c + 1.0) * res
    i = s.astype(jnp.int32)
    return i, s - i.astype(jnp.float32)


def _hash_base(xi, yi, zi):
    h = (xi * _c32(_P1) + yi) * _c32(_P2) + zi
    return h * _c32(_P3)


def _sc_body(xs_h, ys_h, zs_h, tab_h, t14_h, t15_h, out_h,
             tab_v, x0_v, y0_v, z0_v, x1_v, y1_v, z1_v,
             i14_0, i15_0, i14_1, i15_1, r14_0, r15_0, r14_1, r15_1, out_v,
             s14_0, s15_0, s14_1, s15_1, semo, *, batch):
    pw = batch // _NW
    nblk = pw // _P
    wid = lax.axis_index("s") * _NC + lax.axis_index("c")
    pltpu.sync_copy(tab_h, tab_v)
    lanes = lax.iota(jnp.int32, 16)
    nvec = _P // 16

    bufs = [
        dict(x=x0_v, y=y0_v, z=z0_v, i14=i14_0, i15=i15_0, r14=r14_0,
             r15=r15_0, s14=s14_0, s15=s15_0),
        dict(x=x1_v, y=y1_v, z=z1_v, i14=i14_1, i15=i15_1, r14=r14_1,
             r15=r15_1, s14=s14_1, s15=s15_1),
    ]

    def base_of(blk):
        return wid * jnp.int32(pw) + blk * jnp.int32(_P)

    def prefetch(blk, b):
        pt0 = base_of(blk)
        pltpu.sync_copy(xs_h.at[pl.ds(pt0, _P)], b["x"])
        pltpu.sync_copy(ys_h.at[pl.ds(pt0, _P)], b["y"])
        pltpu.sync_copy(zs_h.at[pl.ds(pt0, _P)], b["z"])

        def hash_body(v, c2):
            off = v * jnp.int32(16)
            x = b["x"][pl.ds(off, 16)]
            y = b["y"][pl.ds(off, 16)]
            z = b["z"][pl.ds(off, 16)]
            xi, _ = _floor_frac(x, 512.0)
            yi, _ = _floor_frac(y, 512.0)
            zi, _ = _floor_frac(z, 512.0)
            h = _hash_base(xi, yi, zi)
            for c in range(8):
                hc = h if c == 0 else h + _c32(_OFFS[c])
                b["i14"][pl.ds(off + jnp.int32(c * _P), 16)] = (
                    lax.shift_right_logical(hc & jnp.int32(2 ** 15 - 1),
                                            jnp.int32(2)))
                b["i15"][pl.ds(off + jnp.int32(c * _P), 16)] = (
                    lax.shift_right_logical(hc & jnp.int32(2 ** 16 - 1),
                                            jnp.int32(2)))
            return c2

        lax.fori_loop(jnp.int32(0), jnp.int32(nvec), hash_body, jnp.int32(0))
        for j in range(_NCH):
            o = j * _CH
            pltpu.async_copy(t14_h.at[b["i14"].at[pl.ds(o, _CH)]],
                             b["r14"].at[pl.ds(o, _CH)], b["s14"])
            pltpu.async_copy(t15_h.at[b["i15"].at[pl.ds(o, _CH)]],
                             b["r15"].at[pl.ds(o, _CH)], b["s15"])

    def compute(blk, b, par):
        pt0 = base_of(blk)
        parw = jnp.int32(par * _P * 32)

        def main_body(v, c2):
            off = v * jnp.int32(16)
            x = b["x"][pl.ds(off, 16)]
            y = b["y"][pl.ds(off, 16)]
            z = b["z"][pl.ds(off, 16)]
            rowv = (off + lanes) * jnp.int32(32) + parw

            for l in range(5):
                res = float(16 << l)
                xi, tx = _floor_frac(x, res)
                yi, ty = _floor_frac(y, res)
                zi, tz = _floor_frac(z, res)
                h = _hash_base(xi, yi, zi)
                wk = _weights(tx, ty, tz)
                mask = jnp.int32(2 ** (l + 1) - 1)
                a0 = a1 = None
                for c in range(8):
                    hc = h if c == 0 else h + _c32(_OFFS[c])
                    q = hc & mask
                    f0 = plsc.load_gather(tab_v, [q + jnp.int32(_BASE0[l])])
                    f1 = plsc.load_gather(tab_v, [q + jnp.int32(_BASE1[l])])
                    a0 = f0 * wk[c] if a0 is None else a0 + f0 * wk[c]
                    a1 = f1 * wk[c] if a1 is None else a1 + f1 * wk[c]
                plsc.store_scatter(out_v, [rowv + jnp.int32(2 * l)], a0)
                plsc.store_scatter(out_v, [rowv + jnp.int32(2 * l + 1)], a1)

            xi, tx = _floor_frac(x, 512.0)
            yi, ty = _floor_frac(y, 512.0)
            zi, tz = _floor_frac(z, 512.0)
            h = _hash_base(xi, yi, zi)
            wk = _weights(tx, ty, tz)
            hcs = [h if c == 0 else h + _c32(_OFFS[c]) for c in range(8)]
            for l in range(5, _N_VMEM_LEV):
                mask = jnp.int32(2 ** (l + 1) - 1)
                a0 = a1 = None
                for c in range(8):
                    q = hcs[c] & mask
                    f0 = plsc.load_gather(tab_v, [q + jnp.int32(_BASE0[l])])
                    f1 = plsc.load_gather(tab_v, [q + jnp.int32(_BASE1[l])])
                    a0 = f0 * wk[c] if a0 is None else a0 + f0 * wk[c]
                    a1 = f1 * wk[c] if a1 is None else a1 + f1 * wk[c]
                plsc.store_scatter(out_v, [rowv + jnp.int32(2 * l)], a0)
                plsc.store_scatter(out_v, [rowv + jnp.int32(2 * l + 1)], a1)
            return c2

        lax.fori_loop(jnp.int32(0), jnp.int32(nvec), main_body, jnp.int32(0))

        pltpu.make_async_copy(t14_h.at[pl.ds(0, 8 * _P)], b["r14"],
                              b["s14"]).wait()
        pltpu.make_async_copy(t15_h.at[pl.ds(0, 8 * _P)], b["r15"],
                              b["s15"]).wait()

        def tail_body(v, c2):
            off = v * jnp.int32(16)
            x = b["x"][pl.ds(off, 16)]
            y = b["y"][pl.ds(off, 16)]
            z = b["z"][pl.ds(off, 16)]
            xi, tx = _floor_frac(x, 512.0)
            yi, ty = _floor_frac(y, 512.0)
            zi, tz = _floor_frac(z, 512.0)
            h = _hash_base(xi, yi, zi)
            wk = _weights(tx, ty, tz)
            hcs = [h if c == 0 else h + _c32(_OFFS[c]) for c in range(8)]
            j0s = [(hc & jnp.int32(3)) * jnp.int32(2) for hc in hcs]
            rowv = (off + lanes) * jnp.int32(32) + parw
            pos = off + lanes
            for r_v, col in ((b["r14"], 28), (b["r15"], 30)):
                a0 = a1 = None
                for c in range(8):
                    fp = pos + jnp.int32(c * _P)
                    f0 = plsc.load_gather(r_v, [fp, j0s[c]])
                    f1 = plsc.load_gather(r_v, [fp, j0s[c] + jnp.int32(1)])
                    a0 = f0 * wk[c] if a0 is None else a0 + f0 * wk[c]
                    a1 = f1 * wk[c] if a1 is None else a1 + f1 * wk[c]
                plsc.store_scatter(out_v, [rowv + jnp.int32(col)], a0)
                plsc.store_scatter(out_v, [rowv + jnp.int32(col + 1)], a1)
            return c2

        lax.fori_loop(jnp.int32(0), jnp.int32(nvec), tail_body, jnp.int32(0))

        pltpu.async_copy(out_v.at[pl.ds(par * _P * 32, _P * 32)],
                         out_h.at[pl.ds(pt0 * jnp.int32(32), _P * 32)], semo)

    prefetch(jnp.int32(0), bufs[0])

    def pair_body(g, carry):
        for sub in range(2):
            blk = g * jnp.int32(2) + jnp.int32(sub)
            par = sub
            nxt = blk + jnp.int32(1)

            @pl.when(blk >= jnp.int32(2))
            def _():
                pltpu.make_async_copy(out_v.at[pl.ds(0, _P * 32)],
                                      out_h.at[pl.ds(0, _P * 32)], semo).wait()

            @pl.when(nxt < jnp.int32(nblk))
            def _():
                prefetch(nxt, bufs[1 - par])

            compute(blk, bufs[par], par)
        return carry

    lax.fori_loop(jnp.int32(0), jnp.int32(nblk // 2), pair_body, jnp.int32(0))
    for _ in range(2):
        pltpu.make_async_copy(out_v.at[pl.ds(0, _P * 32)],
                              out_h.at[pl.ds(0, _P * 32)], semo).wait()


def kernel(coords, tables):
    batch = coords.shape[0]
    assert batch % (_NW * _P) == 0 and (batch // (_NW * _P)) % 2 == 0
    coords = coords.astype(jnp.float32)
    xyz = coords.T
    xs, ys, zs = xyz[0], xyz[1], xyz[2]

    parts = []
    for l in range(_N_VMEM_LEV):
        t = tables[l].astype(jnp.float32)
        parts.append(t[:, 0])
        parts.append(t[:, 1])
    parts.append(jnp.zeros((4,), jnp.float32))
    tab_lo = jnp.concatenate(parts)

    mesh = plsc.VectorSubcoreMesh(core_axis_name="c", subcore_axis_name="s")
    run = pl.kernel(
        functools.partial(_sc_body, batch=batch),
        out_type=jax.ShapeDtypeStruct((batch * 32,), jnp.float32),
        mesh=mesh,
        compiler_params=pltpu.CompilerParams(needs_layout_passes=False,
                                             use_tc_tiling_on_sc=False),
        scratch_types=[
            pltpu.VMEM((_TAB_WORDS,), jnp.float32),
            pltpu.VMEM((_P,), jnp.float32),
            pltpu.VMEM((_P,), jnp.float32),
            pltpu.VMEM((_P,), jnp.float32),
            pltpu.VMEM((_P,), jnp.float32),
            pltpu.VMEM((_P,), jnp.float32),
            pltpu.VMEM((_P,), jnp.float32),
            pltpu.VMEM((8 * _P,), jnp.int32),
            pltpu.VMEM((8 * _P,), jnp.int32),
            pltpu.VMEM((8 * _P,), jnp.int32),
            pltpu.VMEM((8 * _P,), jnp.int32),
            pltpu.VMEM((8 * _P, 8), jnp.float32),
            pltpu.VMEM((8 * _P, 8), jnp.float32),
            pltpu.VMEM((8 * _P, 8), jnp.float32),
            pltpu.VMEM((8 * _P, 8), jnp.float32),
            pltpu.VMEM((2 * _P * 32,), jnp.float32),
            pltpu.SemaphoreType.DMA,
            pltpu.SemaphoreType.DMA,
            pltpu.SemaphoreType.DMA,
            pltpu.SemaphoreType.DMA,
            pltpu.SemaphoreType.DMA,
        ],
    )
    t14r = tables[14].astype(jnp.float32).reshape(-1, 8)
    t15r = tables[15].astype(jnp.float32).reshape(-1, 8)
    out = run(xs, ys, zs, tab_lo, t14r, t15r)
    return out.reshape(batch, 32)


# speedup vs baseline: 1.0817x; 1.0817x over previous
import functools
import numpy as np
import jax
import jax.numpy as jnp
from jax import lax
from jax.experimental import pallas as pl
from jax.experimental.pallas import tpu as pltpu
from jax.experimental.pallas import tpu_sc as plsc

_NC, _NS = 2, 16
_NW = _NC * _NS
_P = 256
_CH = 512
_NCH = 8 * _P // _CH

_P1, _P2, _P3 = 2654435761, 29675113, 123456789


def _c32(v):
    return jnp.int32(np.int32(np.uint32(v & 0xFFFFFFFF)))


_HA = (_P1 * _P2 * _P3) & 0xFFFFFFFF
_HB = (_P2 * _P3) & 0xFFFFFFFF
_HC = _P3 & 0xFFFFFFFF
_OFFS = [((k & 1) * _HA + ((k >> 1) & 1) * _HB + ((k >> 2) & 1) * _HC) & 0xFFFFFFFF
         for k in range(8)]

_N_VMEM_LEV = 14
_BASE0 = [2 ** (l + 2) - 4 for l in range(16)]
_BASE1 = [_BASE0[l] + 2 ** (l + 1) for l in range(16)]
_TAB_WORDS = 2 ** 16


def _weights(tx, ty, tz):
    u = 1.0 - tx
    v = 1.0 - ty
    w = 1.0 - tz
    w00 = u * v
    w10 = tx * v
    w01 = u * ty
    w11 = tx * ty
    return [w00 * w, w10 * w, w01 * w, w11 * w,
            w00 * tz, w10 * tz, w01 * tz, w11 * tz]


def _floor_frac(c, res):
    s = (c + 1.0) * res
    i = s.astype(jnp.int32)
    return i, s - i.astype(jnp.float32)


def _hash_base(xi, yi, zi):
    h = (xi * _c32(_P1) + yi) * _c32(_P2) + zi
    return h * _c32(_P3)


def _sc_body(xs_h, ys_h, zs_h, tab_h, t14_h, t15_h, out_h,
             tab_v, x_v, y_v, z_v, i14_v, i15_v, r14_v, r15_v, out_v,
             sem14, sem15, semo, *, batch):
    pw = batch // _NW
    nblk = pw // _P
    wid = lax.axis_index("s") * _NC + lax.axis_index("c")
    pltpu.sync_copy(tab_h, tab_v)
    lanes = lax.iota(jnp.int32, 16)
    zeros16 = jnp.zeros((16,), jnp.int32)
    ones16 = jnp.ones((16,), jnp.int32)

    def block_body(blk, carry):
        pt0 = wid * jnp.int32(pw) + blk * jnp.int32(_P)
        pltpu.sync_copy(xs_h.at[pl.ds(pt0, _P)], x_v)
        pltpu.sync_copy(ys_h.at[pl.ds(pt0, _P)], y_v)
        pltpu.sync_copy(zs_h.at[pl.ds(pt0, _P)], z_v)
        par = (blk & jnp.int32(1)) * jnp.int32(_P * 32)

        @pl.when(blk >= jnp.int32(2))
        def _():
            pltpu.make_async_copy(out_v.at[pl.ds(0, _P * 32)],
                                  out_h.at[pl.ds(0, _P * 32)], semo).wait()

        def hash_body(v, c2):
            off = v * jnp.int32(16)
            x = x_v[pl.ds(off, 16)]
            y = y_v[pl.ds(off, 16)]
            z = z_v[pl.ds(off, 16)]
            xi, _ = _floor_frac(x, 512.0)
            yi, _ = _floor_frac(y, 512.0)
            zi, _ = _floor_frac(z, 512.0)
            h = _hash_base(xi, yi, zi)
            for c in range(8):
                hc = h if c == 0 else h + _c32(_OFFS[c])
                i14_v[pl.ds(off + jnp.int32(c * _P), 16)] = lax.shift_right_logical(
                    hc & jnp.int32(2 ** 15 - 1), jnp.int32(2))
                i15_v[pl.ds(off + jnp.int32(c * _P), 16)] = lax.shift_right_logical(
                    hc & jnp.int32(2 ** 16 - 1), jnp.int32(2))
            return c2

        lax.fori_loop(jnp.int32(0), jnp.int32(_P // 16), hash_body, jnp.int32(0))

        handles = []
        for j in range(_NCH):
            o = j * _CH
            handles.append(pltpu.async_copy(
                t14_h.at[i14_v.at[pl.ds(o, _CH)]],
                r14_v.at[pl.ds(o, _CH)], sem14))
            handles.append(pltpu.async_copy(
                t15_h.at[i15_v.at[pl.ds(o, _CH)]],
                r15_v.at[pl.ds(o, _CH)], sem15))

        def main_body(v, c2):
            off = v * jnp.int32(16)
            x = x_v[pl.ds(off, 16)]
            y = y_v[pl.ds(off, 16)]
            z = z_v[pl.ds(off, 16)]
            rowv = (off + lanes) * jnp.int32(32) + par

            for l in range(5):
                res = float(16 << l)
                xi, tx = _floor_frac(x, res)
                yi, ty = _floor_frac(y, res)
                zi, tz = _floor_frac(z, res)
                h = _hash_base(xi, yi, zi)
                wk = _weights(tx, ty, tz)
                mask = jnp.int32(2 ** (l + 1) - 1)
                a0 = a1 = None
                for c in range(8):
                    hc = h if c == 0 else h + _c32(_OFFS[c])
                    q = hc & mask
                    f0 = plsc.load_gather(tab_v, [q + jnp.int32(_BASE0[l])])
                    f1 = plsc.load_gather(tab_v, [q + jnp.int32(_BASE1[l])])
                    a0 = f0 * wk[c] if a0 is None else a0 + f0 * wk[c]
                    a1 = f1 * wk[c] if a1 is None else a1 + f1 * wk[c]
                plsc.store_scatter(out_v, [rowv + jnp.int32(2 * l)], a0)
                plsc.store_scatter(out_v, [rowv + jnp.int32(2 * l + 1)], a1)

            xi, tx = _floor_frac(x, 512.0)
            yi, ty = _floor_frac(y, 512.0)
            zi, tz = _floor_frac(z, 512.0)
            h = _hash_base(xi, yi, zi)
            wk = _weights(tx, ty, tz)
            hcs = [h if c == 0 else h + _c32(_OFFS[c]) for c in range(8)]
            for l in range(5, _N_VMEM_LEV):
                mask = jnp.int32(2 ** (l + 1) - 1)
                a0 = a1 = None
                for c in range(8):
                    q = hcs[c] & mask
                    f0 = plsc.load_gather(tab_v, [q + jnp.int32(_BASE0[l])])
                    f1 = plsc.load_gather(tab_v, [q + jnp.int32(_BASE1[l])])
                    a0 = f0 * wk[c] if a0 is None else a0 + f0 * wk[c]
                    a1 = f1 * wk[c] if a1 is None else a1 + f1 * wk[c]
                plsc.store_scatter(out_v, [rowv + jnp.int32(2 * l)], a0)
                plsc.store_scatter(out_v, [rowv + jnp.int32(2 * l + 1)], a1)
            return c2

        lax.fori_loop(jnp.int32(0), jnp.int32(_P // 16), main_body, jnp.int32(0))

        for h in handles:
            h.wait()

        def tail_body(v, c2):
            off = v * jnp.int32(16)
            x = x_v[pl.ds(off, 16)]
            y = y_v[pl.ds(off, 16)]
            z = z_v[pl.ds(off, 16)]
            xi, tx = _floor_frac(x, 512.0)
            yi, ty = _floor_frac(y, 512.0)
            zi, tz = _floor_frac(z, 512.0)
            h = _hash_base(xi, yi, zi)
            wk = _weights(tx, ty, tz)
            hcs = [h if c == 0 else h + _c32(_OFFS[c]) for c in range(8)]
            j0s = [(hc & jnp.int32(3)) * jnp.int32(2) for hc in hcs]
            rowv = (off + lanes) * jnp.int32(32) + par
            pos = off + lanes
            for lvl, (r_v, col) in ((14, (r14_v, 28)), (15, (r15_v, 30))):
                a0 = a1 = None
                for c in range(8):
                    fp = pos + jnp.int32(c * _P)
                    f0 = plsc.load_gather(r_v, [fp, j0s[c]])
                    f1 = plsc.load_gather(r_v, [fp, j0s[c] + jnp.int32(1)])
                    a0 = f0 * wk[c] if a0 is None else a0 + f0 * wk[c]
                    a1 = f1 * wk[c] if a1 is None else a1 + f1 * wk[c]
                plsc.store_scatter(out_v, [rowv + jnp.int32(col)], a0)
                plsc.store_scatter(out_v, [rowv + jnp.int32(col + 1)], a1)
            return c2

        lax.fori_loop(jnp.int32(0), jnp.int32(_P // 16), tail_body, jnp.int32(0))

        pltpu.async_copy(out_v.at[pl.ds(par, _P * 32)],
                         out_h.at[pl.ds(pt0 * jnp.int32(32), _P * 32)], semo)
        return carry

    lax.fori_loop(jnp.int32(0), jnp.int32(nblk), block_body, jnp.int32(0))
    for _ in range(2):
        pltpu.make_async_copy(out_v.at[pl.ds(0, _P * 32)],
                              out_h.at[pl.ds(0, _P * 32)], semo).wait()


def kernel(coords, tables):
    batch = coords.shape[0]
    assert batch % (_NW * _P) == 0
    coords = coords.astype(jnp.float32)
    xyz = coords.T
    xs, ys, zs = xyz[0], xyz[1], xyz[2]

    parts = []
    for l in range(_N_VMEM_LEV):
        t = tables[l].astype(jnp.float32)
        parts.append(t[:, 0])
        parts.append(t[:, 1])
    parts.append(jnp.zeros((4,), jnp.float32))
    tab_lo = jnp.concatenate(parts)

    mesh = plsc.VectorSubcoreMesh(core_axis_name="c", subcore_axis_name="s")
    run = pl.kernel(
        functools.partial(_sc_body, batch=batch),
        out_type=jax.ShapeDtypeStruct((batch * 32,), jnp.float32),
        mesh=mesh,
        compiler_params=pltpu.CompilerParams(needs_layout_passes=False, use_tc_tiling_on_sc=False),
        scratch_types=[
            pltpu.VMEM((_TAB_WORDS,), jnp.float32),
            pltpu.VMEM((_P,), jnp.float32),
            pltpu.VMEM((_P,), jnp.float32),
            pltpu.VMEM((_P,), jnp.float32),
            pltpu.VMEM((8 * _P,), jnp.int32),
            pltpu.VMEM((8 * _P,), jnp.int32),
            pltpu.VMEM((8 * _P, 8), jnp.float32),
            pltpu.VMEM((8 * _P, 8), jnp.float32),
            pltpu.VMEM((2 * _P * 32,), jnp.float32),
            pltpu.SemaphoreType.DMA,
            pltpu.SemaphoreType.DMA,
            pltpu.SemaphoreType.DMA,
        ],
    )
    t14r = tables[14].astype(jnp.float32).reshape(-1, 8)
    t15r = tables[15].astype(jnp.float32).reshape(-1, 8)
    out = run(xs, ys, zs, tab_lo, t14r, t15r)
    return out.reshape(batch, 32)
